# baseline (speedup 1.0000x reference)
import jax
import jax.numpy as jnp
from jax import lax
from jax.experimental import pallas as pl
from jax.experimental.pallas import tpu as pltpu
from jax.experimental.pallas import tpu_sc as plsc

_N = 10000
_D = 128
_E = 320000

_NSUB = 16
_NCORE = 2
_ROWS_PER_SUB = 640
_ACC = _NSUB * _ROWS_PER_SUB
_CHUNK = 128
_NCHUNK = 2560
_PAD_E = _NCHUNK * _CHUNK
_CPS = _NCHUNK // (_NCORE * _NSUB)
_C0N = 160
_C1N = 0
_BLK = 1000


def _sc_scatter(zb, rc, zmb):
  mesh = plsc.VectorSubcoreMesh(core_axis_name="c", subcore_axis_name="s")
  nslot = 4
  c0n, c1n = _C0N, _C1N
  cmax = max(c0n, c1n)

  def body(zb_hbm, rc_hbm, zmb_hbm, sp_hbm, slab,
           mb0, mb1, mb2, mb3, acc,
           gs0, gs1, gs2, gs3, ss0, ss1, ss2, ss3):
    mb = [mb0, mb1, mb2, mb3]
    gs = [gs0, gs1, gs2, gs3]
    ss = [ss0, ss1, ss2, ss3]
    c = lax.axis_index("c")
    s = lax.axis_index("s")
    ncs = jnp.where(c == 0, c0n, c1n)
    ngrp = ncs // nslot
    base = jnp.where(c == 0, s * c0n, _NSUB * c0n + s * c1n)

    def gather(k, i):
      pltpu.async_copy(zb_hbm.at[slab.at[k, 0]], mb[i], gs[i])

    def wait_gather(k, i):
      pltpu.make_async_copy(zb_hbm.at[slab.at[k, 0]], mb[i], gs[i]).wait()

    def scatter(k, i):
      pltpu.async_copy(mb[i], acc.at[slab.at[k, 1]], ss[i], add=True)

    def wait_scatter(k, i):
      pltpu.make_async_copy(mb[i], acc.at[slab.at[k, 1]], ss[i]).wait()

    @pl.loop(0, _ROWS_PER_SUB // _CHUNK)
    def _(k5):
      pltpu.sync_copy(zmb_hbm, acc.at[pl.ds(s * _ROWS_PER_SUB + k5 * _CHUNK, _CHUNK)])
    @pl.when(ncs > 0)
    def _():
      pltpu.sync_copy(rc_hbm.at[pl.ds(base, cmax)], slab)
    plsc.subcore_barrier()

    @pl.when(ncs > 0)
    def _():
      gather(0, 0)
      gather(1, 1)

      @pl.loop(0, ngrp)
      def _(t):
        k4 = t * nslot
        for i in range(2):
          @pl.when(t > 0)
          def _():
            wait_scatter(k4 - 2 + i, 2 + i)
          gather(k4 + 2 + i, 2 + i)
        for i in range(2):
          wait_gather(k4 + i, i)
          scatter(k4 + i, i)
        for i in range(2):
          @pl.when(t < ngrp - 1)
          def _():
            wait_scatter(k4 + i, i)
            gather(k4 + 4 + i, i)
        for i in range(2):
          wait_gather(k4 + 2 + i, 2 + i)
          scatter(k4 + 2 + i, 2 + i)

      for i in range(2):
        wait_scatter(ncs - 4 + i, i)
        wait_scatter(ncs - 2 + i, 2 + i)
        plsc.subcore_barrier()

    @pl.loop(0, _ROWS_PER_SUB // _CHUNK)
    def _(k5):
      off = s * _ROWS_PER_SUB + k5 * _CHUNK
      pltpu.sync_copy(acc.at[pl.ds(off, _CHUNK)],
                      sp_hbm.at[pl.ds(c * _ACC + off, _CHUNK)])

  f = pl.kernel(
      body,
      out_type=jax.ShapeDtypeStruct((_NCORE * _ACC, _D), jnp.bfloat16),
      mesh=mesh,
      compiler_params=pltpu.CompilerParams(use_tc_tiling_on_sc=False),
      scratch_types=(
          [pltpu.VMEM((max(_C0N, _C1N), 2, _CHUNK), jnp.int32)]
          + [pltpu.VMEM((_CHUNK, _D), jnp.bfloat16) for _ in range(4)]
          + [pltpu.VMEM_SHARED((_ACC, _D), jnp.bfloat16)]
          + [pltpu.SemaphoreType.DMA for _ in range(8)]
      ),
  )
  return f(zb, rc, zmb)


def _sc_degree(rc, z16, o16):
  mesh = plsc.VectorSubcoreMesh(core_axis_name="c", subcore_axis_name="s")
  cpw = _NCHUNK // (_NCORE * _NSUB)

  def body(rc_hbm, z16_hbm, o16_hbm, dg_hbm, cbuf, ones_v, dacc):
    c = lax.axis_index("c")
    s = lax.axis_index("s")
    w = c * _NSUB + s

    @pl.loop(0, _ROWS_PER_SUB // _CHUNK)
    def _(k5):
      pltpu.sync_copy(z16_hbm, dacc.at[pl.ds(s * _ROWS_PER_SUB + k5 * _CHUNK, _CHUNK)])
    pltpu.sync_copy(o16_hbm, ones_v)
    pltpu.sync_copy(rc_hbm.at[pl.ds(w * cpw, cpw)], cbuf)
    plsc.subcore_barrier()

    @pl.loop(0, cpw)
    def _(k):
      pltpu.sync_copy(ones_v, dacc.at[cbuf.at[k, 1]], add=True)

    plsc.subcore_barrier()

    @pl.loop(0, _ROWS_PER_SUB // _CHUNK)
    def _(k5):
      off = s * _ROWS_PER_SUB + k5 * _CHUNK
      pltpu.sync_copy(dacc.at[pl.ds(off, _CHUNK)],
                      dg_hbm.at[pl.ds(c * _ACC + off, _CHUNK)])

  f = pl.kernel(
      body,
      out_type=jax.ShapeDtypeStruct((_NCORE * _ACC, 16), jnp.float32),
      mesh=mesh,
      compiler_params=pltpu.CompilerParams(use_tc_tiling_on_sc=False),
      scratch_types=[
          pltpu.VMEM((cpw, 2, _CHUNK), jnp.int32),
          pltpu.VMEM((_CHUNK, 16), jnp.float32),
          pltpu.VMEM_SHARED((_ACC, 16), jnp.float32),
      ],
  )
  return f(rc, z16, o16)


def _tc_first(x, wcat, b):
  def f(x_ref, w_ref, b_ref, zb_ref, z_ref, r_ref):
    y = jnp.dot(x_ref[...], w_ref[...], preferred_element_type=jnp.float32)
    z = y[:, :_D]
    zb_ref[...] = z.astype(jnp.bfloat16)
    z_ref[...] = z
    r_ref[...] = y[:, _D:] + b_ref[...]

  return pl.pallas_call(
      f,
      grid=(_N // _BLK,),
      in_specs=[
          pl.BlockSpec((_BLK, _D), lambda i: (i, 0)),
          pl.BlockSpec((_D, 2 * _D), lambda i: (0, 0)),
          pl.BlockSpec((1, _D), lambda i: (0, 0)),
      ],
      out_specs=[pl.BlockSpec((_BLK, _D), lambda i: (i, 0))] * 3,
      out_shape=[
          jax.ShapeDtypeStruct((_N, _D), jnp.bfloat16),
          jax.ShapeDtypeStruct((_N, _D), jnp.float32),
          jax.ShapeDtypeStruct((_N, _D), jnp.float32),
      ],
  )(x, wcat, b.reshape(1, _D))


def _tc_mid(sp, dg, z, r, wcat, b):
  def f(sp_ref, dg_ref, z_ref, r_ref, w_ref, b_ref, zb_ref, zo_ref, ro_ref):
    ssum = (sp_ref[0].astype(jnp.float32) + sp_ref[1].astype(jnp.float32)
            + z_ref[...])
    deg = 1.0 + dg_ref[0][:, 0:1] + dg_ref[1][:, 0:1]
    h = jnp.maximum(ssum / deg + r_ref[...], 0.0)
    y = jnp.dot(h, w_ref[...], preferred_element_type=jnp.float32)
    z = y[:, :_D]
    zb_ref[...] = z.astype(jnp.bfloat16)
    zo_ref[...] = z
    ro_ref[...] = y[:, _D:] + b_ref[...]

  return pl.pallas_call(
      f,
      grid=(_N // _BLK,),
      in_specs=[
          pl.BlockSpec((_NCORE, _BLK, _D), lambda i: (0, i, 0)),
          pl.BlockSpec((_NCORE, _BLK, 16), lambda i: (0, i, 0)),
          pl.BlockSpec((_BLK, _D), lambda i: (i, 0)),
          pl.BlockSpec((_BLK, _D), lambda i: (i, 0)),
          pl.BlockSpec((_D, 2 * _D), lambda i: (0, 0)),
          pl.BlockSpec((1, _D), lambda i: (0, 0)),
      ],
      out_specs=[pl.BlockSpec((_BLK, _D), lambda i: (i, 0))] * 3,
      out_shape=[
          jax.ShapeDtypeStruct((_N, _D), jnp.bfloat16),
          jax.ShapeDtypeStruct((_N, _D), jnp.float32),
          jax.ShapeDtypeStruct((_N, _D), jnp.float32),
      ],
  )(sp, dg, z, r, wcat, b.reshape(1, _D))


def _tc_final(sp, dg, z, r):
  def f(sp_ref, dg_ref, z_ref, r_ref, o_ref):
    ssum = (sp_ref[0].astype(jnp.float32) + sp_ref[1].astype(jnp.float32)
            + z_ref[...])
    deg = 1.0 + dg_ref[0][:, 0:1] + dg_ref[1][:, 0:1]
    o_ref[...] = jax.nn.sigmoid(jnp.maximum(ssum / deg + r_ref[...], 0.0))

  return pl.pallas_call(
      f,
      grid=(_N // _BLK,),
      in_specs=[
          pl.BlockSpec((_NCORE, _BLK, _D), lambda i: (0, i, 0)),
          pl.BlockSpec((_NCORE, _BLK, 16), lambda i: (0, i, 0)),
          pl.BlockSpec((_BLK, _D), lambda i: (i, 0)),
          pl.BlockSpec((_BLK, _D), lambda i: (i, 0)),
      ],
      out_specs=pl.BlockSpec((_BLK, _D), lambda i: (i, 0)),
      out_shape=jax.ShapeDtypeStruct((_N, _D), jnp.float32),
  )(sp, dg, z, r)


def kernel(x, edge_index, W_out0, b_out0, W_root0,
           W_out1, b_out1, W_root1, W_out2, b_out2, W_root2):
  row = edge_index[0]
  col = edge_index[1]
  pad = _PAD_E - _E
  row2d = jnp.concatenate([row, jnp.zeros((pad,), jnp.int32)]).reshape(_NCHUNK, _CHUNK)
  col2d = jnp.concatenate([col, jnp.full((pad,), _N, jnp.int32)]).reshape(_NCHUNK, _CHUNK)
  rc = jnp.stack([row2d, col2d], axis=1)
  zmb = jnp.zeros((_CHUNK, _D), jnp.bfloat16)
  z16 = jnp.zeros((_CHUNK, 16), jnp.float32)
  o16 = jnp.ones((_CHUNK, 16), jnp.float32)

  wcat0 = jnp.concatenate([W_out0, W_root0], axis=1)
  wcat1 = jnp.concatenate([W_out1, W_root1], axis=1)
  wcat2 = jnp.concatenate([W_out2, W_root2], axis=1)

  dg = _sc_degree(rc, z16, o16).reshape(_NCORE, _ACC, 16)

  zb1, z1, r1 = _tc_first(x, wcat0, b_out0)
  sp1 = _sc_scatter(zb1, rc, zmb).reshape(_NCORE, _ACC, _D)

  zb2, z2, r2 = _tc_mid(sp1, dg, z1, r1, wcat1, b_out1)
  sp2 = _sc_scatter(zb2, rc, zmb).reshape(_NCORE, _ACC, _D)

  zb3, z3, r3 = _tc_mid(sp2, dg, z2, r2, wcat2, b_out2)
  sp3 = _sc_scatter(zb3, rc, zmb).reshape(_NCORE, _ACC, _D)

  return _tc_final(sp3, dg, z3, r3)


# speedup vs baseline: 1.0378x; 1.0378x over previous
import jax
import jax.numpy as jnp
from jax import lax
from jax.experimental import pallas as pl
from jax.experimental.pallas import tpu as pltpu
from jax.experimental.pallas import tpu_sc as plsc

_N = 10000
_D = 128
_E = 320000

_NSUB = 16
_NCORE = 2
_ROWS_PER_SUB = 640
_ACC = _NSUB * _ROWS_PER_SUB
_CHUNK = 64
_NCHUNK = 5120
_PAD_E = _NCHUNK * _CHUNK
_CPS = _NCHUNK // (_NCORE * _NSUB)
_C0N = 160
_C1N = 160
_BLK = 1000


def _sc_scatter(zb, rc, zmb):
  mesh = plsc.VectorSubcoreMesh(core_axis_name="c", subcore_axis_name="s")
  nslot = 4
  c0n, c1n = _C0N, _C1N
  cmax = max(c0n, c1n)

  def body(zb_hbm, rc_hbm, zmb_hbm, sp_hbm, slab,
           mb0, mb1, mb2, mb3, acc,
           gs0, gs1, gs2, gs3, ss0, ss1, ss2, ss3):
    mb = [mb0, mb1, mb2, mb3]
    gs = [gs0, gs1, gs2, gs3]
    ss = [ss0, ss1, ss2, ss3]
    c = lax.axis_index("c")
    s = lax.axis_index("s")
    ncs = jnp.where(c == 0, c0n, c1n)
    ngrp = ncs // nslot
    base = jnp.where(c == 0, s * c0n, _NSUB * c0n + s * c1n)

    def gather(k, i):
      pltpu.async_copy(zb_hbm.at[slab.at[k, 0]], mb[i], gs[i])

    def wait_gather(k, i):
      pltpu.make_async_copy(zb_hbm.at[slab.at[k, 0]], mb[i], gs[i]).wait()

    def scatter(k, i):
      pltpu.async_copy(mb[i], acc.at[slab.at[k, 1]], ss[i], add=True)

    def wait_scatter(k, i):
      pltpu.make_async_copy(mb[i], acc.at[slab.at[k, 1]], ss[i]).wait()

    @pl.loop(0, _ROWS_PER_SUB // _CHUNK)
    def _(k5):
      pltpu.sync_copy(zmb_hbm, acc.at[pl.ds(s * _ROWS_PER_SUB + k5 * _CHUNK, _CHUNK)])
    @pl.when(ncs > 0)
    def _():
      pltpu.sync_copy(rc_hbm.at[pl.ds(base, cmax)], slab)
    plsc.subcore_barrier()

    @pl.when(ncs > 0)
    def _():
      gather(0, 0)
      gather(1, 1)

      @pl.loop(0, ngrp)
      def _(t):
        k4 = t * nslot
        for i in range(2):
          @pl.when(t > 0)
          def _():
            wait_scatter(k4 - 2 + i, 2 + i)
          gather(k4 + 2 + i, 2 + i)
        for i in range(2):
          wait_gather(k4 + i, i)
          scatter(k4 + i, i)
        for i in range(2):
          @pl.when(t < ngrp - 1)
          def _():
            wait_scatter(k4 + i, i)
            gather(k4 + 4 + i, i)
        for i in range(2):
          wait_gather(k4 + 2 + i, 2 + i)
          scatter(k4 + 2 + i, 2 + i)

      for i in range(2):
        wait_scatter(ncs - 4 + i, i)
        wait_scatter(ncs - 2 + i, 2 + i)
        plsc.subcore_barrier()

    @pl.loop(0, _ROWS_PER_SUB // _CHUNK)
    def _(k5):
      off = s * _ROWS_PER_SUB + k5 * _CHUNK
      pltpu.sync_copy(acc.at[pl.ds(off, _CHUNK)],
                      sp_hbm.at[pl.ds(c * _ACC + off, _CHUNK)])

  f = pl.kernel(
      body,
      out_type=jax.ShapeDtypeStruct((_NCORE * _ACC, _D), jnp.bfloat16),
      mesh=mesh,
      compiler_params=pltpu.CompilerParams(use_tc_tiling_on_sc=False),
      scratch_types=(
          [pltpu.VMEM((max(_C0N, _C1N), 2, _CHUNK), jnp.int32)]
          + [pltpu.VMEM((_CHUNK, _D), jnp.bfloat16) for _ in range(4)]
          + [pltpu.VMEM_SHARED((_ACC, _D), jnp.bfloat16)]
          + [pltpu.SemaphoreType.DMA for _ in range(8)]
      ),
  )
  return f(zb, rc, zmb)


def _sc_degree(rc, z16, o16):
  mesh = plsc.VectorSubcoreMesh(core_axis_name="c", subcore_axis_name="s")
  cpw = _NCHUNK // (_NCORE * _NSUB)

  def body(rc_hbm, z16_hbm, o16_hbm, dg_hbm, cbuf, ones_v, dacc):
    c = lax.axis_index("c")
    s = lax.axis_index("s")
    w = c * _NSUB + s

    @pl.loop(0, _ROWS_PER_SUB // _CHUNK)
    def _(k5):
      pltpu.sync_copy(z16_hbm, dacc.at[pl.ds(s * _ROWS_PER_SUB + k5 * _CHUNK, _CHUNK)])
    pltpu.sync_copy(o16_hbm, ones_v)
    pltpu.sync_copy(rc_hbm.at[pl.ds(w * cpw, cpw)], cbuf)
    plsc.subcore_barrier()

    @pl.loop(0, cpw)
    def _(k):
      pltpu.sync_copy(ones_v, dacc.at[cbuf.at[k, 1]], add=True)

    plsc.subcore_barrier()

    @pl.loop(0, _ROWS_PER_SUB // _CHUNK)
    def _(k5):
      off = s * _ROWS_PER_SUB + k5 * _CHUNK
      pltpu.sync_copy(dacc.at[pl.ds(off, _CHUNK)],
                      dg_hbm.at[pl.ds(c * _ACC + off, _CHUNK)])

  f = pl.kernel(
      body,
      out_type=jax.ShapeDtypeStruct((_NCORE * _ACC, 16), jnp.float32),
      mesh=mesh,
      compiler_params=pltpu.CompilerParams(use_tc_tiling_on_sc=False),
      scratch_types=[
          pltpu.VMEM((cpw, 2, _CHUNK), jnp.int32),
          pltpu.VMEM((_CHUNK, 16), jnp.float32),
          pltpu.VMEM_SHARED((_ACC, 16), jnp.float32),
      ],
  )
  return f(rc, z16, o16)


def _tc_first(x, wcat, b):
  def f(x_ref, w_ref, b_ref, zb_ref, z_ref, r_ref):
    y = jnp.dot(x_ref[...], w_ref[...], preferred_element_type=jnp.float32)
    z = y[:, :_D]
    zb_ref[...] = z.astype(jnp.bfloat16)
    z_ref[...] = z
    r_ref[...] = y[:, _D:] + b_ref[...]

  return pl.pallas_call(
      f,
      grid=(_N // _BLK,),
      in_specs=[
          pl.BlockSpec((_BLK, _D), lambda i: (i, 0)),
          pl.BlockSpec((_D, 2 * _D), lambda i: (0, 0)),
          pl.BlockSpec((1, _D), lambda i: (0, 0)),
      ],
      out_specs=[pl.BlockSpec((_BLK, _D), lambda i: (i, 0))] * 3,
      out_shape=[
          jax.ShapeDtypeStruct((_N, _D), jnp.bfloat16),
          jax.ShapeDtypeStruct((_N, _D), jnp.float32),
          jax.ShapeDtypeStruct((_N, _D), jnp.float32),
      ],
  )(x, wcat, b.reshape(1, _D))


def _tc_mid(sp, dg, z, r, wcat, b):
  def f(sp_ref, dg_ref, z_ref, r_ref, w_ref, b_ref, zb_ref, zo_ref, ro_ref):
    ssum = (sp_ref[0].astype(jnp.float32) + sp_ref[1].astype(jnp.float32)
            + z_ref[...])
    deg = 1.0 + dg_ref[0][:, 0:1] + dg_ref[1][:, 0:1]
    h = jnp.maximum(ssum / deg + r_ref[...], 0.0)
    y = jnp.dot(h, w_ref[...], preferred_element_type=jnp.float32)
    z = y[:, :_D]
    zb_ref[...] = z.astype(jnp.bfloat16)
    zo_ref[...] = z
    ro_ref[...] = y[:, _D:] + b_ref[...]

  return pl.pallas_call(
      f,
      grid=(_N // _BLK,),
      in_specs=[
          pl.BlockSpec((_NCORE, _BLK, _D), lambda i: (0, i, 0)),
          pl.BlockSpec((_NCORE, _BLK, 16), lambda i: (0, i, 0)),
          pl.BlockSpec((_BLK, _D), lambda i: (i, 0)),
          pl.BlockSpec((_BLK, _D), lambda i: (i, 0)),
          pl.BlockSpec((_D, 2 * _D), lambda i: (0, 0)),
          pl.BlockSpec((1, _D), lambda i: (0, 0)),
      ],
      out_specs=[pl.BlockSpec((_BLK, _D), lambda i: (i, 0))] * 3,
      out_shape=[
          jax.ShapeDtypeStruct((_N, _D), jnp.bfloat16),
          jax.ShapeDtypeStruct((_N, _D), jnp.float32),
          jax.ShapeDtypeStruct((_N, _D), jnp.float32),
      ],
  )(sp, dg, z, r, wcat, b.reshape(1, _D))


def _tc_final(sp, dg, z, r):
  def f(sp_ref, dg_ref, z_ref, r_ref, o_ref):
    ssum = (sp_ref[0].astype(jnp.float32) + sp_ref[1].astype(jnp.float32)
            + z_ref[...])
    deg = 1.0 + dg_ref[0][:, 0:1] + dg_ref[1][:, 0:1]
    o_ref[...] = jax.nn.sigmoid(jnp.maximum(ssum / deg + r_ref[...], 0.0))

  return pl.pallas_call(
      f,
      grid=(_N // _BLK,),
      in_specs=[
          pl.BlockSpec((_NCORE, _BLK, _D), lambda i: (0, i, 0)),
          pl.BlockSpec((_NCORE, _BLK, 16), lambda i: (0, i, 0)),
          pl.BlockSpec((_BLK, _D), lambda i: (i, 0)),
          pl.BlockSpec((_BLK, _D), lambda i: (i, 0)),
      ],
      out_specs=pl.BlockSpec((_BLK, _D), lambda i: (i, 0)),
      out_shape=jax.ShapeDtypeStruct((_N, _D), jnp.float32),
  )(sp, dg, z, r)


def kernel(x, edge_index, W_out0, b_out0, W_root0,
           W_out1, b_out1, W_root1, W_out2, b_out2, W_root2):
  row = edge_index[0]
  col = edge_index[1]
  pad = _PAD_E - _E
  row2d = jnp.concatenate([row, jnp.zeros((pad,), jnp.int32)]).reshape(_NCHUNK, _CHUNK)
  col2d = jnp.concatenate([col, jnp.full((pad,), _N, jnp.int32)]).reshape(_NCHUNK, _CHUNK)
  rc = jnp.stack([row2d, col2d], axis=1)
  zmb = jnp.zeros((_CHUNK, _D), jnp.bfloat16)
  z16 = jnp.zeros((_CHUNK, 16), jnp.float32)
  o16 = jnp.ones((_CHUNK, 16), jnp.float32)

  wcat0 = jnp.concatenate([W_out0, W_root0], axis=1)
  wcat1 = jnp.concatenate([W_out1, W_root1], axis=1)
  wcat2 = jnp.concatenate([W_out2, W_root2], axis=1)

  dg = _sc_degree(rc, z16, o16).reshape(_NCORE, _ACC, 16)

  zb1, z1, r1 = _tc_first(x, wcat0, b_out0)
  sp1 = _sc_scatter(zb1, rc, zmb).reshape(_NCORE, _ACC, _D)

  zb2, z2, r2 = _tc_mid(sp1, dg, z1, r1, wcat1, b_out1)
  sp2 = _sc_scatter(zb2, rc, zmb).reshape(_NCORE, _ACC, _D)

  zb3, z3, r3 = _tc_mid(sp2, dg, z2, r2, wcat2, b_out2)
  sp3 = _sc_scatter(zb3, rc, zmb).reshape(_NCORE, _ACC, _D)

  return _tc_final(sp3, dg, z3, r3)


# speedup vs baseline: 1.1029x; 1.0628x over previous
import jax
import jax.numpy as jnp
from jax import lax
from jax.experimental import pallas as pl
from jax.experimental.pallas import tpu as pltpu
from jax.experimental.pallas import tpu_sc as plsc

_N = 10000
_D = 128
_E = 320000

_NSUB = 16
_NCORE = 2
_ROWS_PER_SUB = 640
_ACC = _NSUB * _ROWS_PER_SUB
_CHUNK = 128
_NCHUNK = 2560
_PAD_E = _NCHUNK * _CHUNK
_CPS = _NCHUNK // (_NCORE * _NSUB)
_C0N = 136
_C1N = 24
_BLK = 1000


def _sc_scatter(zb, rc, zmb):
  mesh = plsc.VectorSubcoreMesh(core_axis_name="c", subcore_axis_name="s")
  nslot = 4
  c0n, c1n = _C0N, _C1N
  cmax = max(c0n, c1n)

  def body(zb_hbm, rc_hbm, zmb_hbm, sp_hbm, slab,
           mb0, mb1, mb2, mb3, acc,
           gs0, gs1, gs2, gs3, ss0, ss1, ss2, ss3):
    mb = [mb0, mb1, mb2, mb3]
    gs = [gs0, gs1, gs2, gs3]
    ss = [ss0, ss1, ss2, ss3]
    c = lax.axis_index("c")
    s = lax.axis_index("s")
    ncs = jnp.where(c == 0, c0n, c1n)
    ngrp = ncs // nslot
    base = jnp.where(c == 0, s * c0n, _NSUB * c0n + s * c1n)

    def gather(k, i):
      pltpu.async_copy(zb_hbm.at[slab.at[k, 0]], mb[i], gs[i])

    def wait_gather(k, i):
      pltpu.make_async_copy(zb_hbm.at[slab.at[k, 0]], mb[i], gs[i]).wait()

    def scatter(k, i):
      pltpu.async_copy(mb[i], acc.at[slab.at[k, 1]], ss[i], add=True)

    def wait_scatter(k, i):
      pltpu.make_async_copy(mb[i], acc.at[slab.at[k, 1]], ss[i]).wait()

    @pl.loop(0, _ROWS_PER_SUB // _CHUNK)
    def _(k5):
      pltpu.sync_copy(zmb_hbm, acc.at[pl.ds(s * _ROWS_PER_SUB + k5 * _CHUNK, _CHUNK)])
    @pl.when(ncs > 0)
    def _():
      pltpu.sync_copy(rc_hbm.at[pl.ds(base, cmax)], slab)
    plsc.subcore_barrier()

    @pl.when(ncs > 0)
    def _():
      gather(0, 0)
      gather(1, 1)

      @pl.loop(0, ngrp)
      def _(t):
        k4 = t * nslot
        for i in range(2):
          @pl.when(t > 0)
          def _():
            wait_scatter(k4 - 2 + i, 2 + i)
          gather(k4 + 2 + i, 2 + i)
        for i in range(2):
          wait_gather(k4 + i, i)
          scatter(k4 + i, i)
        for i in range(2):
          @pl.when(t < ngrp - 1)
          def _():
            wait_scatter(k4 + i, i)
            gather(k4 + 4 + i, i)
        for i in range(2):
          wait_gather(k4 + 2 + i, 2 + i)
          scatter(k4 + 2 + i, 2 + i)

      for i in range(2):
        wait_scatter(ncs - 4 + i, i)
        wait_scatter(ncs - 2 + i, 2 + i)
        plsc.subcore_barrier()

    @pl.loop(0, _ROWS_PER_SUB // _CHUNK)
    def _(k5):
      off = s * _ROWS_PER_SUB + k5 * _CHUNK
      pltpu.sync_copy(acc.at[pl.ds(off, _CHUNK)],
                      sp_hbm.at[pl.ds(c * _ACC + off, _CHUNK)])

  f = pl.kernel(
      body,
      out_type=jax.ShapeDtypeStruct((_NCORE * _ACC, _D), jnp.bfloat16),
      mesh=mesh,
      compiler_params=pltpu.CompilerParams(use_tc_tiling_on_sc=False),
      scratch_types=(
          [pltpu.VMEM((max(_C0N, _C1N), 2, _CHUNK), jnp.int32)]
          + [pltpu.VMEM((_CHUNK, _D), jnp.bfloat16) for _ in range(4)]
          + [pltpu.VMEM_SHARED((_ACC, _D), jnp.bfloat16)]
          + [pltpu.SemaphoreType.DMA for _ in range(8)]
      ),
  )
  return f(zb, rc, zmb)


def _sc_degree(rc, z16, o16):
  mesh = plsc.VectorSubcoreMesh(core_axis_name="c", subcore_axis_name="s")
  cpw = _NCHUNK // (_NCORE * _NSUB)

  def body(rc_hbm, z16_hbm, o16_hbm, dg_hbm, cbuf, ones_v, dacc):
    c = lax.axis_index("c")
    s = lax.axis_index("s")
    w = c * _NSUB + s

    @pl.loop(0, _ROWS_PER_SUB // _CHUNK)
    def _(k5):
      pltpu.sync_copy(z16_hbm, dacc.at[pl.ds(s * _ROWS_PER_SUB + k5 * _CHUNK, _CHUNK)])
    pltpu.sync_copy(o16_hbm, ones_v)
    pltpu.sync_copy(rc_hbm.at[pl.ds(w * cpw, cpw)], cbuf)
    plsc.subcore_barrier()

    @pl.loop(0, cpw)
    def _(k):
      pltpu.sync_copy(ones_v, dacc.at[cbuf.at[k, 1]], add=True)

    plsc.subcore_barrier()

    @pl.loop(0, _ROWS_PER_SUB // _CHUNK)
    def _(k5):
      off = s * _ROWS_PER_SUB + k5 * _CHUNK
      pltpu.sync_copy(dacc.at[pl.ds(off, _CHUNK)],
                      dg_hbm.at[pl.ds(c * _ACC + off, _CHUNK)])

  f = pl.kernel(
      body,
      out_type=jax.ShapeDtypeStruct((_NCORE * _ACC, 16), jnp.float32),
      mesh=mesh,
      compiler_params=pltpu.CompilerParams(use_tc_tiling_on_sc=False),
      scratch_types=[
          pltpu.VMEM((cpw, 2, _CHUNK), jnp.int32),
          pltpu.VMEM((_CHUNK, 16), jnp.float32),
          pltpu.VMEM_SHARED((_ACC, 16), jnp.float32),
      ],
  )
  return f(rc, z16, o16)


def _tc_first(x, wcat, b):
  def f(x_ref, w_ref, b_ref, zb_ref, z_ref, r_ref):
    y = jnp.dot(x_ref[...], w_ref[...], preferred_element_type=jnp.float32)
    z = y[:, :_D]
    zb_ref[...] = z.astype(jnp.bfloat16)
    z_ref[...] = z
    r_ref[...] = y[:, _D:] + b_ref[...]

  return pl.pallas_call(
      f,
      grid=(_N // _BLK,),
      in_specs=[
          pl.BlockSpec((_BLK, _D), lambda i: (i, 0)),
          pl.BlockSpec((_D, 2 * _D), lambda i: (0, 0)),
          pl.BlockSpec((1, _D), lambda i: (0, 0)),
      ],
      out_specs=[pl.BlockSpec((_BLK, _D), lambda i: (i, 0))] * 3,
      out_shape=[
          jax.ShapeDtypeStruct((_N, _D), jnp.bfloat16),
          jax.ShapeDtypeStruct((_N, _D), jnp.float32),
          jax.ShapeDtypeStruct((_N, _D), jnp.float32),
      ],
  )(x, wcat, b.reshape(1, _D))


def _tc_mid(sp, dg, z, r, wcat, b):
  def f(sp_ref, dg_ref, z_ref, r_ref, w_ref, b_ref, zb_ref, zo_ref, ro_ref):
    ssum = (sp_ref[0].astype(jnp.float32) + sp_ref[1].astype(jnp.float32)
            + z_ref[...])
    deg = 1.0 + dg_ref[0][:, 0:1] + dg_ref[1][:, 0:1]
    h = jnp.maximum(ssum / deg + r_ref[...], 0.0)
    y = jnp.dot(h, w_ref[...], preferred_element_type=jnp.float32)
    z = y[:, :_D]
    zb_ref[...] = z.astype(jnp.bfloat16)
    zo_ref[...] = z
    ro_ref[...] = y[:, _D:] + b_ref[...]

  return pl.pallas_call(
      f,
      grid=(_N // _BLK,),
      in_specs=[
          pl.BlockSpec((_NCORE, _BLK, _D), lambda i: (0, i, 0)),
          pl.BlockSpec((_NCORE, _BLK, 16), lambda i: (0, i, 0)),
          pl.BlockSpec((_BLK, _D), lambda i: (i, 0)),
          pl.BlockSpec((_BLK, _D), lambda i: (i, 0)),
          pl.BlockSpec((_D, 2 * _D), lambda i: (0, 0)),
          pl.BlockSpec((1, _D), lambda i: (0, 0)),
      ],
      out_specs=[pl.BlockSpec((_BLK, _D), lambda i: (i, 0))] * 3,
      out_shape=[
          jax.ShapeDtypeStruct((_N, _D), jnp.bfloat16),
          jax.ShapeDtypeStruct((_N, _D), jnp.float32),
          jax.ShapeDtypeStruct((_N, _D), jnp.float32),
      ],
  )(sp, dg, z, r, wcat, b.reshape(1, _D))


def _tc_final(sp, dg, z, r):
  def f(sp_ref, dg_ref, z_ref, r_ref, o_ref):
    ssum = (sp_ref[0].astype(jnp.float32) + sp_ref[1].astype(jnp.float32)
            + z_ref[...])
    deg = 1.0 + dg_ref[0][:, 0:1] + dg_ref[1][:, 0:1]
    o_ref[...] = jax.nn.sigmoid(jnp.maximum(ssum / deg + r_ref[...], 0.0))

  return pl.pallas_call(
      f,
      grid=(_N // _BLK,),
      in_specs=[
          pl.BlockSpec((_NCORE, _BLK, _D), lambda i: (0, i, 0)),
          pl.BlockSpec((_NCORE, _BLK, 16), lambda i: (0, i, 0)),
          pl.BlockSpec((_BLK, _D), lambda i: (i, 0)),
          pl.BlockSpec((_BLK, _D), lambda i: (i, 0)),
      ],
      out_specs=pl.BlockSpec((_BLK, _D), lambda i: (i, 0)),
      out_shape=jax.ShapeDtypeStruct((_N, _D), jnp.float32),
  )(sp, dg, z, r)


def kernel(x, edge_index, W_out0, b_out0, W_root0,
           W_out1, b_out1, W_root1, W_out2, b_out2, W_root2):
  row = edge_index[0]
  col = edge_index[1]
  pad = _PAD_E - _E
  row2d = jnp.concatenate([row, jnp.zeros((pad,), jnp.int32)]).reshape(_NCHUNK, _CHUNK)
  col2d = jnp.concatenate([col, jnp.full((pad,), _N, jnp.int32)]).reshape(_NCHUNK, _CHUNK)
  rc = jnp.stack([row2d, col2d], axis=1)
  zmb = jnp.zeros((_CHUNK, _D), jnp.bfloat16)
  z16 = jnp.zeros((_CHUNK, 16), jnp.float32)
  o16 = jnp.ones((_CHUNK, 16), jnp.float32)

  wcat0 = jnp.concatenate([W_out0, W_root0], axis=1)
  wcat1 = jnp.concatenate([W_out1, W_root1], axis=1)
  wcat2 = jnp.concatenate([W_out2, W_root2], axis=1)

  dg = _sc_degree(rc, z16, o16).reshape(_NCORE, _ACC, 16)

  zb1, z1, r1 = _tc_first(x, wcat0, b_out0)
  sp1 = _sc_scatter(zb1, rc, zmb).reshape(_NCORE, _ACC, _D)

  zb2, z2, r2 = _tc_mid(sp1, dg, z1, r1, wcat1, b_out1)
  sp2 = _sc_scatter(zb2, rc, zmb).reshape(_NCORE, _ACC, _D)

  zb3, z3, r3 = _tc_mid(sp2, dg, z2, r2, wcat2, b_out2)
  sp3 = _sc_scatter(zb3, rc, zmb).reshape(_NCORE, _ACC, _D)

  return _tc_final(sp3, dg, z3, r3)


# speedup vs baseline: 1.1296x; 1.0242x over previous
import jax
import jax.numpy as jnp
from jax import lax
from jax.experimental import pallas as pl
from jax.experimental.pallas import tpu as pltpu
from jax.experimental.pallas import tpu_sc as plsc

_N = 10000
_D = 128
_E = 320000

_NSUB = 16
_NCORE = 2
_ROWS_PER_SUB = 640
_ACC = _NSUB * _ROWS_PER_SUB
_CHUNK = 128
_NCHUNK = 2560
_PAD_E = _NCHUNK * _CHUNK
_CPS = _NCHUNK // (_NCORE * _NSUB)
_C0N = 120
_C1N = 40
_BLK = 1000


def _sc_scatter(zb, rc, zmb):
  mesh = plsc.VectorSubcoreMesh(core_axis_name="c", subcore_axis_name="s")
  nslot = 4
  c0n, c1n = _C0N, _C1N
  cmax = max(c0n, c1n)

  def body(zb_hbm, rc_hbm, zmb_hbm, sp_hbm, slab,
           mb0, mb1, mb2, mb3, acc,
           gs0, gs1, gs2, gs3, ss0, ss1, ss2, ss3):
    mb = [mb0, mb1, mb2, mb3]
    gs = [gs0, gs1, gs2, gs3]
    ss = [ss0, ss1, ss2, ss3]
    c = lax.axis_index("c")
    s = lax.axis_index("s")
    ncs = jnp.where(c == 0, c0n, c1n)
    ngrp = ncs // nslot
    base = jnp.where(c == 0, s * c0n, _NSUB * c0n + s * c1n)

    def gather(k, i):
      pltpu.async_copy(zb_hbm.at[slab.at[k, 0]], mb[i], gs[i])

    def wait_gather(k, i):
      pltpu.make_async_copy(zb_hbm.at[slab.at[k, 0]], mb[i], gs[i]).wait()

    def scatter(k, i):
      pltpu.async_copy(mb[i], acc.at[slab.at[k, 1]], ss[i], add=True)

    def wait_scatter(k, i):
      pltpu.make_async_copy(mb[i], acc.at[slab.at[k, 1]], ss[i]).wait()

    @pl.loop(0, _ROWS_PER_SUB // _CHUNK)
    def _(k5):
      pltpu.sync_copy(zmb_hbm, acc.at[pl.ds(s * _ROWS_PER_SUB + k5 * _CHUNK, _CHUNK)])
    pltpu.sync_copy(rc_hbm.at[pl.ds(base, cmax)], slab)
    plsc.subcore_barrier()

    gather(0, 0)
    gather(1, 1)

    @pl.loop(0, ngrp)
    def _(t):
      k4 = t * nslot
      for i in range(2):
        @pl.when(t > 0)
        def _():
          wait_scatter(k4 - 2 + i, 2 + i)
        gather(k4 + 2 + i, 2 + i)
      for i in range(2):
        wait_gather(k4 + i, i)
        scatter(k4 + i, i)
      for i in range(2):
        @pl.when(t < ngrp - 1)
        def _():
          wait_scatter(k4 + i, i)
          gather(k4 + 4 + i, i)
      for i in range(2):
        wait_gather(k4 + 2 + i, 2 + i)
        scatter(k4 + 2 + i, 2 + i)

    for i in range(2):
      wait_scatter(ncs - 4 + i, i)
      wait_scatter(ncs - 2 + i, 2 + i)
    plsc.subcore_barrier()

    @pl.loop(0, _ROWS_PER_SUB // _CHUNK)
    def _(k5):
      off = s * _ROWS_PER_SUB + k5 * _CHUNK
      pltpu.sync_copy(acc.at[pl.ds(off, _CHUNK)],
                      sp_hbm.at[pl.ds(c * _ACC + off, _CHUNK)])

  f = pl.kernel(
      body,
      out_type=jax.ShapeDtypeStruct((_NCORE * _ACC, _D), jnp.bfloat16),
      mesh=mesh,
      compiler_params=pltpu.CompilerParams(use_tc_tiling_on_sc=False),
      scratch_types=(
          [pltpu.VMEM((max(_C0N, _C1N), 2, _CHUNK), jnp.int32)]
          + [pltpu.VMEM((_CHUNK, _D), jnp.bfloat16) for _ in range(4)]
          + [pltpu.VMEM_SHARED((_ACC, _D), jnp.bfloat16)]
          + [pltpu.SemaphoreType.DMA for _ in range(8)]
      ),
  )
  return f(zb, rc, zmb)


def _sc_degree(rc, z16, o16):
  mesh = plsc.VectorSubcoreMesh(core_axis_name="c", subcore_axis_name="s")
  cpw = _NCHUNK // (_NCORE * _NSUB)

  def body(rc_hbm, z16_hbm, o16_hbm, dg_hbm, cbuf, ones_v, dacc):
    c = lax.axis_index("c")
    s = lax.axis_index("s")
    w = c * _NSUB + s

    @pl.loop(0, _ROWS_PER_SUB // _CHUNK)
    def _(k5):
      pltpu.sync_copy(z16_hbm, dacc.at[pl.ds(s * _ROWS_PER_SUB + k5 * _CHUNK, _CHUNK)])
    pltpu.sync_copy(o16_hbm, ones_v)
    pltpu.sync_copy(rc_hbm.at[pl.ds(w * cpw, cpw)], cbuf)
    plsc.subcore_barrier()

    @pl.loop(0, cpw)
    def _(k):
      pltpu.sync_copy(ones_v, dacc.at[cbuf.at[k, 1]], add=True)

    plsc.subcore_barrier()

    @pl.loop(0, _ROWS_PER_SUB // _CHUNK)
    def _(k5):
      off = s * _ROWS_PER_SUB + k5 * _CHUNK
      pltpu.sync_copy(dacc.at[pl.ds(off, _CHUNK)],
                      dg_hbm.at[pl.ds(c * _ACC + off, _CHUNK)])

  f = pl.kernel(
      body,
      out_type=jax.ShapeDtypeStruct((_NCORE * _ACC, 16), jnp.float32),
      mesh=mesh,
      compiler_params=pltpu.CompilerParams(use_tc_tiling_on_sc=False),
      scratch_types=[
          pltpu.VMEM((cpw, 2, _CHUNK), jnp.int32),
          pltpu.VMEM((_CHUNK, 16), jnp.float32),
          pltpu.VMEM_SHARED((_ACC, 16), jnp.float32),
      ],
  )
  return f(rc, z16, o16)


def _tc_first(x, wcat, b):
  def f(x_ref, w_ref, b_ref, zb_ref, z_ref, r_ref):
    y = jnp.dot(x_ref[...], w_ref[...], preferred_element_type=jnp.float32)
    z = y[:, :_D]
    zb_ref[...] = z.astype(jnp.bfloat16)
    z_ref[...] = z
    r_ref[...] = y[:, _D:] + b_ref[...]

  return pl.pallas_call(
      f,
      grid=(_N // _BLK,),
      in_specs=[
          pl.BlockSpec((_BLK, _D), lambda i: (i, 0)),
          pl.BlockSpec((_D, 2 * _D), lambda i: (0, 0)),
          pl.BlockSpec((1, _D), lambda i: (0, 0)),
      ],
      out_specs=[pl.BlockSpec((_BLK, _D), lambda i: (i, 0))] * 3,
      out_shape=[
          jax.ShapeDtypeStruct((_N, _D), jnp.bfloat16),
          jax.ShapeDtypeStruct((_N, _D), jnp.float32),
          jax.ShapeDtypeStruct((_N, _D), jnp.float32),
      ],
  )(x, wcat, b.reshape(1, _D))


def _tc_mid(sp, dg, z, r, wcat, b):
  def f(sp_ref, dg_ref, z_ref, r_ref, w_ref, b_ref, zb_ref, zo_ref, ro_ref):
    ssum = (sp_ref[0].astype(jnp.float32) + sp_ref[1].astype(jnp.float32)
            + z_ref[...])
    deg = 1.0 + dg_ref[0][:, 0:1] + dg_ref[1][:, 0:1]
    h = jnp.maximum(ssum / deg + r_ref[...], 0.0)
    y = jnp.dot(h, w_ref[...], preferred_element_type=jnp.float32)
    z = y[:, :_D]
    zb_ref[...] = z.astype(jnp.bfloat16)
    zo_ref[...] = z
    ro_ref[...] = y[:, _D:] + b_ref[...]

  return pl.pallas_call(
      f,
      grid=(_N // _BLK,),
      in_specs=[
          pl.BlockSpec((_NCORE, _BLK, _D), lambda i: (0, i, 0)),
          pl.BlockSpec((_NCORE, _BLK, 16), lambda i: (0, i, 0)),
          pl.BlockSpec((_BLK, _D), lambda i: (i, 0)),
          pl.BlockSpec((_BLK, _D), lambda i: (i, 0)),
          pl.BlockSpec((_D, 2 * _D), lambda i: (0, 0)),
          pl.BlockSpec((1, _D), lambda i: (0, 0)),
      ],
      out_specs=[pl.BlockSpec((_BLK, _D), lambda i: (i, 0))] * 3,
      out_shape=[
          jax.ShapeDtypeStruct((_N, _D), jnp.bfloat16),
          jax.ShapeDtypeStruct((_N, _D), jnp.float32),
          jax.ShapeDtypeStruct((_N, _D), jnp.float32),
      ],
  )(sp, dg, z, r, wcat, b.reshape(1, _D))


def _tc_final(sp, dg, z, r):
  def f(sp_ref, dg_ref, z_ref, r_ref, o_ref):
    ssum = (sp_ref[0].astype(jnp.float32) + sp_ref[1].astype(jnp.float32)
            + z_ref[...])
    deg = 1.0 + dg_ref[0][:, 0:1] + dg_ref[1][:, 0:1]
    o_ref[...] = jax.nn.sigmoid(jnp.maximum(ssum / deg + r_ref[...], 0.0))

  return pl.pallas_call(
      f,
      grid=(_N // _BLK,),
      in_specs=[
          pl.BlockSpec((_NCORE, _BLK, _D), lambda i: (0, i, 0)),
          pl.BlockSpec((_NCORE, _BLK, 16), lambda i: (0, i, 0)),
          pl.BlockSpec((_BLK, _D), lambda i: (i, 0)),
          pl.BlockSpec((_BLK, _D), lambda i: (i, 0)),
      ],
      out_specs=pl.BlockSpec((_BLK, _D), lambda i: (i, 0)),
      out_shape=jax.ShapeDtypeStruct((_N, _D), jnp.float32),
  )(sp, dg, z, r)


def kernel(x, edge_index, W_out0, b_out0, W_root0,
           W_out1, b_out1, W_root1, W_out2, b_out2, W_root2):
  row = edge_index[0]
  col = edge_index[1]
  pad = _PAD_E - _E
  row2d = jnp.concatenate([row, jnp.zeros((pad,), jnp.int32)]).reshape(_NCHUNK, _CHUNK)
  col2d = jnp.concatenate([col, jnp.full((pad,), _N, jnp.int32)]).reshape(_NCHUNK, _CHUNK)
  rc = jnp.stack([row2d, col2d], axis=1)
  zmb = jnp.zeros((_CHUNK, _D), jnp.bfloat16)
  z16 = jnp.zeros((_CHUNK, 16), jnp.float32)
  o16 = jnp.ones((_CHUNK, 16), jnp.float32)

  wcat0 = jnp.concatenate([W_out0, W_root0], axis=1)
  wcat1 = jnp.concatenate([W_out1, W_root1], axis=1)
  wcat2 = jnp.concatenate([W_out2, W_root2], axis=1)

  dg = _sc_degree(rc, z16, o16).reshape(_NCORE, _ACC, 16)

  zb1, z1, r1 = _tc_first(x, wcat0, b_out0)
  sp1 = _sc_scatter(zb1, rc, zmb).reshape(_NCORE, _ACC, _D)

  zb2, z2, r2 = _tc_mid(sp1, dg, z1, r1, wcat1, b_out1)
  sp2 = _sc_scatter(zb2, rc, zmb).reshape(_NCORE, _ACC, _D)

  zb3, z3, r3 = _tc_mid(sp2, dg, z2, r2, wcat2, b_out2)
  sp3 = _sc_scatter(zb3, rc, zmb).reshape(_NCORE, _ACC, _D)

  return _tc_final(sp3, dg, z3, r3)
